# Initial kernel scaffold; baseline (speedup 1.0000x reference)
import jax
import jax.numpy as jnp
from jax.experimental import pallas as pl


def kernel(x, edge_index, W1l, b1l, W1r, b1r, W2l, b2l, W2r, b2r, Wp1, bp1, Wp2, bp2):
    raise NotImplementedError("write your pallas kernel here")


# speedup vs baseline: 5.1549x; 5.1549x over previous
import functools

import jax
import jax.numpy as jnp
from jax import lax
from jax.experimental import pallas as pl
from jax.experimental.pallas import tpu as pltpu
from jax.experimental.pallas import tpu_sc as plsc

_N = 10000
_E = 320000
_D = 128
_NC = 2
_NS = 16
_NW = _NC * _NS
_EW = _E // _NW
_B = 80
_NB = _EW // _B
_NPAD = 10240
_RPT = _NPAD // _NS
_CW = 16


def _make_seg_sum(with_counts: bool):
    mesh = plsc.VectorSubcoreMesh(core_axis_name="c", subcore_axis_name="s")
    out_type = [jax.ShapeDtypeStruct((_NC, _NPAD, _D), jnp.float32)]
    scratch = [
        pltpu.VMEM_SHARED((_NPAD, _D), jnp.float32),
        pltpu.VMEM((_B,), jnp.int32),
        pltpu.VMEM((_B,), jnp.int32),
        pltpu.VMEM((_B, _D), jnp.float32),
        pltpu.SemaphoreType.DMA,
    ]
    if with_counts:
        out_type.append(jax.ShapeDtypeStruct((_NC, _NPAD), jnp.float32))
        scratch += [
            pltpu.VMEM_SHARED((_NPAD,), jnp.float32),
            pltpu.VMEM((_B,), jnp.float32),
            pltpu.VMEM((_RPT,), jnp.float32),
        ]

    def body(h_hbm, src_hbm, dst_hbm, *rest):
        if with_counts:
            out_hbm, cnt_hbm, acc, sidx, didx, rows, sem, cnt_sp, onesb, zblk = rest
        else:
            out_hbm, acc, sidx, didx, rows, sem = rest
        c = lax.axis_index("c")
        s = lax.axis_index("s")
        wid = c * _NS + s

        z16 = jnp.zeros((16,), jnp.float32)
        o16 = jnp.ones((16,), jnp.float32)

        @pl.loop(0, _B)
        def _zero_rows(i):
            for k in range(_D // 16):
                rows[i, pl.ds(16 * k, 16)] = z16

        r0 = s * _RPT
        if with_counts:
            @pl.loop(0, _RPT // 16)
            def _zero_zblk(i):
                zblk[pl.ds(16 * i, 16)] = z16

            @pl.loop(0, _B // 16)
            def _fill_ones(i):
                onesb[pl.ds(16 * i, 16)] = o16

            pltpu.sync_copy(zblk, cnt_sp.at[pl.ds(r0, _RPT)])

        for k in range(_RPT // _B):
            pltpu.sync_copy(rows, acc.at[pl.ds(r0 + k * _B, _B)])
        plsc.subcore_barrier()

        base = wid * _EW

        @pl.loop(0, _NB)
        def _edge_block(j):
            off = pl.multiple_of(base + j * _B, 8)
            pltpu.sync_copy(src_hbm.at[pl.ds(off, _B)], sidx)
            pltpu.sync_copy(dst_hbm.at[pl.ds(off, _B)], didx)
            pltpu.async_copy(h_hbm.at[sidx], rows, sem).wait()
            pltpu.sync_copy(rows, acc.at[didx], add=True)
            if with_counts:
                pltpu.sync_copy(onesb, cnt_sp.at[didx], add=True)

        plsc.subcore_barrier()
        pltpu.sync_copy(acc.at[pl.ds(r0, _RPT)], out_hbm.at[c, pl.ds(r0, _RPT)])
        if with_counts:
            pltpu.sync_copy(cnt_sp.at[pl.ds(r0, _RPT)],
                            cnt_hbm.at[c, pl.ds(r0, _RPT)])

    return pl.kernel(body, out_type=out_type, mesh=mesh,
                     scratch_types=scratch)


_seg_sum_cnt = _make_seg_sum(True)
_seg_sum = _make_seg_sum(False)

_BR = 1000


def _tc1_body(p_ref, cnt_ref, x_ref, wl_ref, wr_ref, b_ref, o_ref):
    cnt = jnp.sum(cnt_ref[...], axis=1, keepdims=True)
    agg = (p_ref[0] + p_ref[1]) / jnp.clip(cnt, 1.0, None)
    h = (jnp.dot(agg, wl_ref[...], preferred_element_type=jnp.float32)
         + jnp.dot(x_ref[...], wr_ref[...], preferred_element_type=jnp.float32)
         + b_ref[...])
    o_ref[...] = jnp.maximum(h, 0.0)


def _tc2_body(p_ref, cnt_ref, h_ref, w2l_ref, w2r_ref, b2_ref,
              wp1_ref, bp1_ref, wp2_ref, bp2_ref, o_ref):
    cnt = jnp.sum(cnt_ref[...], axis=1, keepdims=True)
    agg = (p_ref[0] + p_ref[1]) / jnp.clip(cnt, 1.0, None)
    z = (jnp.dot(agg, w2l_ref[...], preferred_element_type=jnp.float32)
         + jnp.dot(h_ref[...], w2r_ref[...], preferred_element_type=jnp.float32)
         + b2_ref[...])
    z = jnp.maximum(z, 0.0)
    z = jnp.dot(z, wp1_ref[...], preferred_element_type=jnp.float32) + bp1_ref[...]
    z = jnp.dot(z, wp2_ref[...], preferred_element_type=jnp.float32) + bp2_ref[...]
    m = jnp.max(z, axis=1, keepdims=True)
    z = z - m
    o_ref[...] = z - jnp.log(jnp.sum(jnp.exp(z), axis=1, keepdims=True))


def _row_specs(weight_shapes):
    p_spec = pl.BlockSpec((_NC, _BR, _D), lambda i: (0, i, 0))
    cnt_spec = pl.BlockSpec((_BR, _NC), lambda i: (i, 0))
    x_spec = pl.BlockSpec((_BR, _D), lambda i: (i, 0))
    w_specs = [pl.BlockSpec(s, lambda i: (0, 0)) for s in weight_shapes]
    return [p_spec, cnt_spec, x_spec] + w_specs


_W = (_D, _D)
_BIAS = (1, _D)

_tc1 = pl.pallas_call(
    _tc1_body,
    grid=(_N // _BR,),
    in_specs=_row_specs([_W, _W, _BIAS]),
    out_specs=pl.BlockSpec((_BR, _D), lambda i: (i, 0)),
    out_shape=jax.ShapeDtypeStruct((_N, _D), jnp.float32),
)

_tc2 = pl.pallas_call(
    _tc2_body,
    grid=(_N // _BR,),
    in_specs=_row_specs([_W, _W, _BIAS, _W, _BIAS, _W, _BIAS]),
    out_specs=pl.BlockSpec((_BR, _D), lambda i: (i, 0)),
    out_shape=jax.ShapeDtypeStruct((_N, _D), jnp.float32),
)


def kernel(x, edge_index, W1l, b1l, W1r, b1r, W2l, b2l, W2r, b2r,
           Wp1, bp1, Wp2, bp2):
    src = edge_index[0]
    dst = edge_index[1]
    p1, cnt = _seg_sum_cnt(x, src, dst)
    cnt = cnt.T
    h1 = _tc1(p1, cnt, x, W1l.T, W1r.T, (b1l + b1r)[None, :])
    (p2,) = _seg_sum(h1, src, dst)
    out = _tc2(p2, cnt, h1, W2l.T, W2r.T, (b2l + b2r)[None, :],
               Wp1.T, bp1[None, :], Wp2.T, bp2[None, :])
    return out
